# Initial kernel scaffold; baseline (speedup 1.0000x reference)
import jax
import jax.numpy as jnp
from jax.experimental import pallas as pl


def kernel(points, lengths):
    raise NotImplementedError("write your pallas kernel here")


# speedup vs baseline: 3.7585x; 3.7585x over previous
import functools

import jax
import jax.numpy as jnp
from jax import lax
from jax.experimental import pallas as pl
from jax.experimental.pallas import tpu as pltpu
from jax.experimental.pallas import tpu_sc as plsc

NC, NS, L = 2, 16, 16
B, N, C, K = 16, 8192, 512, 32
CH = C // 2
BUF = 96

BIG = 1 << 30
NEG_INF = float("-inf")
POS_INF = float("inf")

_mesh = plsc.VectorSubcoreMesh(
    core_axis_name="c", subcore_axis_name="s", num_cores=NC, num_subcores=NS
)


def _argmax_resolve(bv, biv):
    mx = jnp.max(bv)
    idxf = jnp.where(bv == mx, biv.astype(jnp.float32), float(BIG))
    return jnp.min(idxf).astype(jnp.int32)


def _my_length(lens_v, b):
    lanes = lax.iota(jnp.int32, L)
    lenf = jnp.where(lanes == b, lens_v[...].astype(jnp.float32), 0.0)
    return jnp.max(lenf).astype(jnp.int32)


def _splat(v):
    return jnp.full((L,), v, jnp.int32)


def _fps_body(pts_hbm, lens_hbm, cen_hbm, px, py, pz, mind, lens_v, idx_v,
              cenx_v, ceny_v, cenz_v):
    cc = lax.axis_index("c")
    ss = lax.axis_index("s")
    wid = ss * NC + cc

    @pl.when(wid < B)
    def _():
        b = wid
        pltpu.sync_copy(pts_hbm.at[pl.ds((b * 3 + 0) * N, N)], px)
        pltpu.sync_copy(pts_hbm.at[pl.ds((b * 3 + 1) * N, N)], py)
        pltpu.sync_copy(pts_hbm.at[pl.ds((b * 3 + 2) * N, N)], pz)
        pltpu.sync_copy(lens_hbm, lens_v)
        lanes = lax.iota(jnp.int32, L)
        lane0 = lanes == 0
        length = _my_length(lens_v, b)
        nv = (length + jnp.int32(L - 1)) // jnp.int32(L)

        bv0 = jnp.full((L,), NEG_INF, jnp.float32)
        biv0 = jnp.zeros((L,), jnp.int32)

        zero = jnp.min(lanes.astype(jnp.float32)).astype(jnp.int32)
        zidx = _splat(zero)
        plsc.store_scatter(idx_v, [zidx], zidx, mask=lane0)

        x0 = plsc.load_gather(px, [zidx])
        y0 = plsc.load_gather(py, [zidx])
        z0 = plsc.load_gather(pz, [zidx])

        def init_i(i, carry):
            bv, biv = carry
            sl = pl.ds(i * L, L)
            dx = px[sl] - x0
            dy = py[sl] - y0
            dz = pz[sl] - z0
            d = dx * dx + dy * dy
            d = d + dz * dz
            mind[sl] = d
            gidx = i * L + lanes
            cand = jnp.where(gidx < length, d, NEG_INF)
            upd = cand > bv
            return jnp.where(upd, cand, bv), jnp.where(upd, gidx, biv)

        bv, biv = lax.fori_loop(0, nv, init_i, (bv0, biv0))
        pick = _argmax_resolve(bv, biv)
        plsc.store_scatter(idx_v, [zidx + 1], _splat(pick), mask=lane0)

        def step(k, pick_prev):
            pv = _splat(pick_prev)
            lx = plsc.load_gather(px, [pv])
            ly = plsc.load_gather(py, [pv])
            lz = plsc.load_gather(pz, [pv])

            def upd_i(i, carry):
                bv, biv = carry
                sl = pl.ds(i * L, L)
                dx = px[sl] - lx
                dy = py[sl] - ly
                dz = pz[sl] - lz
                d = dx * dx + dy * dy
                d = d + dz * dz
                m2 = jnp.minimum(mind[sl], d)
                mind[sl] = m2
                gidx = i * L + lanes
                cand = jnp.where(gidx < length, m2, NEG_INF)
                upd = cand > bv
                return jnp.where(upd, cand, bv), jnp.where(upd, gidx, biv)

            bv, biv = lax.fori_loop(0, nv, upd_i, (bv0, biv0))
            pick_k = _argmax_resolve(bv, biv)
            plsc.store_scatter(idx_v, [_splat(k)], _splat(pick_k), mask=lane0)
            return pick_k

        lax.fori_loop(2, C, step, pick)

        for j in range(C // L):
            sl = pl.ds(j * L, L)
            iv = idx_v[sl]
            cenx_v[sl] = plsc.load_gather(px, [iv])
            ceny_v[sl] = plsc.load_gather(py, [iv])
            cenz_v[sl] = plsc.load_gather(pz, [iv])
        pltpu.sync_copy(cenx_v, cen_hbm.at[pl.ds((b * 3 + 0) * C, C)])
        pltpu.sync_copy(ceny_v, cen_hbm.at[pl.ds((b * 3 + 1) * C, C)])
        pltpu.sync_copy(cenz_v, cen_hbm.at[pl.ds((b * 3 + 2) * C, C)])


def _merge16(ad, ai, bd, bi):
    brd = lax.rev(bd, (0,))
    bri = lax.rev(bi, (0,))
    sel = ad <= brd
    lod = jnp.where(sel, ad, brd)
    loi = jnp.where(sel, ai, bri)
    hid = jnp.where(sel, brd, ad)
    hii = jnp.where(sel, bri, ai)
    lod, loi = plsc.sort_key_val(lod, loi)
    hid, hii = plsc.sort_key_val(hid, hii)
    return (lod, loi), (hid, hii)


def _merge32_low(x, y):
    (x0d, x0i), (x1d, x1i) = x
    (y0d, y0i), (y1d, y1i) = y
    yr0d = lax.rev(y1d, (0,))
    yr0i = lax.rev(y1i, (0,))
    yr1d = lax.rev(y0d, (0,))
    yr1i = lax.rev(y0i, (0,))
    s0 = x0d <= yr0d
    p0d = jnp.where(s0, x0d, yr0d)
    p0i = jnp.where(s0, x0i, yr0i)
    s1 = x1d <= yr1d
    p1d = jnp.where(s1, x1d, yr1d)
    p1i = jnp.where(s1, x1i, yr1i)
    s2 = p0d <= p1d
    b0d = jnp.where(s2, p0d, p1d)
    b0i = jnp.where(s2, p0i, p1i)
    b1d = jnp.where(s2, p1d, p0d)
    b1i = jnp.where(s2, p1i, p0i)
    b0d, b0i = plsc.sort_key_val(b0d, b0i)
    b1d, b1i = plsc.sort_key_val(b1d, b1i)
    return (b0d, b0i), (b1d, b1i)


def _consolidate(bufd, bufi):
    vs = []
    for j in range(BUF // L):
        dj = bufd[pl.ds(j * L, L)]
        ij = bufi[pl.ds(j * L, L)]
        vs.append(plsc.sort_key_val(dj, ij))
    s01 = _merge16(vs[0][0], vs[0][1], vs[1][0], vs[1][1])
    s23 = _merge16(vs[2][0], vs[2][1], vs[3][0], vs[3][1])
    s45 = _merge16(vs[4][0], vs[4][1], vs[5][0], vs[5][1])
    m = _merge32_low(s01, s23)
    m = _merge32_low(m, s45)
    (l0d, l0i), (l1d, l1i) = m
    bufd[pl.ds(0, L)] = l0d
    bufi[pl.ds(0, L)] = l0i
    bufd[pl.ds(L, L)] = l1d
    bufi[pl.ds(L, L)] = l1i
    padd = jnp.full((L,), POS_INF, jnp.float32)
    padi = jnp.full((L,), BIG, jnp.int32)
    for j in range(2, BUF // L):
        bufd[pl.ds(j * L, L)] = padd
        bufi[pl.ds(j * L, L)] = padi
    return jnp.max(l1d)


def _knn_body(pts_hbm, cen_hbm, lens_hbm, grp_hbm,
              px, py, pz, cenx, ceny, cenz, bufd, bufi, lens_v,
              gx_v, gy_v, gz_v):
    cc = lax.axis_index("c")
    ss = lax.axis_index("s")
    wid = ss * NC + cc
    b = wid // 2
    h = wid % 2

    pltpu.sync_copy(pts_hbm.at[pl.ds((b * 3 + 0) * N, N)], px)
    pltpu.sync_copy(pts_hbm.at[pl.ds((b * 3 + 1) * N, N)], py)
    pltpu.sync_copy(pts_hbm.at[pl.ds((b * 3 + 2) * N, N)], pz)
    pltpu.sync_copy(cen_hbm.at[pl.ds((b * 3 + 0) * C + h * CH, CH)], cenx)
    pltpu.sync_copy(cen_hbm.at[pl.ds((b * 3 + 1) * C + h * CH, CH)], ceny)
    pltpu.sync_copy(cen_hbm.at[pl.ds((b * 3 + 2) * C + h * CH, CH)], cenz)
    pltpu.sync_copy(lens_hbm, lens_v)
    lanes = lax.iota(jnp.int32, L)
    length = _my_length(lens_v, b)
    nv = (length + jnp.int32(L - 1)) // jnp.int32(L)

    padd = jnp.full((L,), POS_INF, jnp.float32)
    padi = jnp.full((L,), BIG, jnp.int32)

    def per_center(ci, _):
        civ = _splat(ci)
        cx = plsc.load_gather(cenx, [civ])
        cy = plsc.load_gather(ceny, [civ])
        cz = plsc.load_gather(cenz, [civ])
        for j in range(BUF // L):
            bufd[pl.ds(j * L, L)] = padd
            bufi[pl.ds(j * L, L)] = padi

        def scan_i(i, carry):
            cnt, t = carry
            sl = pl.ds(i * L, L)
            dx = px[sl] - cx
            dy = py[sl] - cy
            dz = pz[sl] - cz
            d = dx * dx + dy * dy
            d = d + dz * dz
            gidx = i * L + lanes
            m = (d < t) & (gidx < length)
            anyhit = jnp.any(m)

            def dohit(cnt, t):
                cum = plsc.cumsum(m.astype(jnp.int32))
                pos = cnt + cum - 1
                plsc.store_scatter(bufd, [pos], d, mask=m)
                plsc.store_scatter(bufi, [pos], gidx, mask=m)
                cnt2 = cnt + jnp.max(cum.astype(jnp.float32)).astype(jnp.int32)

                def consol(_):
                    t2 = _consolidate(bufd, bufi)
                    return jnp.int32(K), t2

                return lax.cond(cnt2 > jnp.int32(BUF - 2 * L), consol,
                                lambda _: (cnt2, t), 0)

            return lax.cond(anyhit, dohit, lambda cnt, t: (cnt, t), cnt, t)

        lax.fori_loop(0, nv, scan_i, (jnp.int32(0), jnp.float32(POS_INF)))
        _consolidate(bufd, bufi)

        for j in range(K // L):
            iv = bufi[pl.ds(j * L, L)]
            vx = plsc.load_gather(px, [iv]) - cx
            vy = plsc.load_gather(py, [iv]) - cy
            vz = plsc.load_gather(pz, [iv]) - cz
            gx_v[pl.ds(ci * K + j * L, L)] = vx
            gy_v[pl.ds(ci * K + j * L, L)] = vy
            gz_v[pl.ds(ci * K + j * L, L)] = vz
        return 0

    lax.fori_loop(0, CH, per_center, 0)
    pltpu.sync_copy(gx_v, grp_hbm.at[pl.ds((b * 3 + 0) * C * K + h * CH * K, CH * K)])
    pltpu.sync_copy(gy_v, grp_hbm.at[pl.ds((b * 3 + 1) * C * K + h * CH * K, CH * K)])
    pltpu.sync_copy(gz_v, grp_hbm.at[pl.ds((b * 3 + 2) * C * K + h * CH * K, CH * K)])


@jax.jit
def kernel(points, lengths):
    pts_t = points.transpose(0, 2, 1).reshape(B * 3 * N)
    lengths = lengths.astype(jnp.int32)

    fps = pl.kernel(
        _fps_body,
        out_type=jax.ShapeDtypeStruct((B * 3 * C,), jnp.float32),
        mesh=_mesh,
        compiler_params=pltpu.CompilerParams(needs_layout_passes=False),
        scratch_types=[
            pltpu.VMEM((N,), jnp.float32),
            pltpu.VMEM((N,), jnp.float32),
            pltpu.VMEM((N,), jnp.float32),
            pltpu.VMEM((N,), jnp.float32),
            pltpu.VMEM((L,), jnp.int32),
            pltpu.VMEM((C,), jnp.int32),
            pltpu.VMEM((C,), jnp.float32),
            pltpu.VMEM((C,), jnp.float32),
            pltpu.VMEM((C,), jnp.float32),
        ],
    )
    cen_t = fps(pts_t, lengths)

    knn = pl.kernel(
        _knn_body,
        out_type=jax.ShapeDtypeStruct((B * 3 * C * K,), jnp.float32),
        mesh=_mesh,
        compiler_params=pltpu.CompilerParams(needs_layout_passes=False),
        scratch_types=[
            pltpu.VMEM((N,), jnp.float32),
            pltpu.VMEM((N,), jnp.float32),
            pltpu.VMEM((N,), jnp.float32),
            pltpu.VMEM((CH,), jnp.float32),
            pltpu.VMEM((CH,), jnp.float32),
            pltpu.VMEM((CH,), jnp.float32),
            pltpu.VMEM((BUF,), jnp.float32),
            pltpu.VMEM((BUF,), jnp.int32),
            pltpu.VMEM((L,), jnp.int32),
            pltpu.VMEM((CH * K,), jnp.float32),
            pltpu.VMEM((CH * K,), jnp.float32),
            pltpu.VMEM((CH * K,), jnp.float32),
        ],
    )
    grp_t = knn(pts_t, cen_t, lengths)

    centers = cen_t.reshape(B, 3, C).transpose(0, 2, 1)
    groups = grp_t.reshape(B, 3, C, K).transpose(0, 2, 3, 1)
    return groups, centers
